# baseline (speedup 1.0000x reference)
import functools

import jax
import jax.numpy as jnp
from jax import lax
from jax.experimental import pallas as pl
from jax.experimental.pallas import tpu as pltpu
from jax.experimental.pallas import tpu_sc as plsc

N = 10000
D = 128
R = 200
E = 320000
SLOPE = (1.0 / 8.0 + 1.0 / 3.0) / 2.0

NPAD = 10240
ROWS_PER_TILE = NPAD // 16
ROW_CHUNK = 128
CH = 128
NW = 32
CPTA = 160
CPTB = 80
EPH = 16 * CPTA * CH
REPK = 256


def _dotT(x, w):
    return lax.dot_general(x, w, (((1,), (1,)), ((), ())),
                           preferred_element_type=jnp.float32)


def _mid_body(pa_ref, pb_ref, norm_ref, ent_ref, wr0_ref, wl0_ref, wl1_ref,
              h1_ref, s1_ref):
    agg = pa_ref[0] + pb_ref[0]
    h1 = _dotT(agg, wr0_ref[...]) * norm_ref[...] + _dotT(ent_ref[...], wl0_ref[...])
    h1 = jnp.where(h1 >= 0, h1, SLOPE * h1)
    h1_ref[...] = h1
    s1_ref[...] = _dotT(h1, wl1_ref[...])


def _mid_tc(p, norm, ent, wr0, wl0, wl1):
    return pl.pallas_call(
        _mid_body,
        grid=(25,),
        in_specs=[
            pl.BlockSpec((1, 400, D), lambda i: (0, i, 0)),
            pl.BlockSpec((1, 400, D), lambda i: (1, i, 0)),
            pl.BlockSpec((400, 1), lambda i: (i, 0)),
            pl.BlockSpec((400, D), lambda i: (i, 0)),
            pl.BlockSpec((D, D), lambda i: (0, 0)),
            pl.BlockSpec((D, D), lambda i: (0, 0)),
            pl.BlockSpec((D, D), lambda i: (0, 0)),
        ],
        out_specs=[
            pl.BlockSpec((400, D), lambda i: (i, 0)),
            pl.BlockSpec((400, D), lambda i: (i, 0)),
        ],
        out_shape=[
            jax.ShapeDtypeStruct((N, D), jnp.float32),
            jax.ShapeDtypeStruct((N, D), jnp.float32),
        ],
    )(p, p, norm, ent, wr0, wl0, wl1)


def _fin_body(pb0_ref, pb1_ref, pa1_ref, norm_ref, s1_ref, wr1_ref, o_ref):
    agg = pb0_ref[0] + pb1_ref[0] + pa1_ref[0]
    h = _dotT(agg, wr1_ref[...]) * norm_ref[...] + s1_ref[...]
    o_ref[...] = jnp.where(h >= 0, h, SLOPE * h)


def _fin_tc(pb, pa, norm, s1, wr1):
    return pl.pallas_call(
        _fin_body,
        grid=(25,),
        in_specs=[
            pl.BlockSpec((1, 400, D), lambda i: (0, i, 0)),
            pl.BlockSpec((1, 400, D), lambda i: (1, i, 0)),
            pl.BlockSpec((1, 400, D), lambda i: (1, i, 0)),
            pl.BlockSpec((400, 1), lambda i: (i, 0)),
            pl.BlockSpec((400, D), lambda i: (i, 0)),
            pl.BlockSpec((D, D), lambda i: (0, 0)),
        ],
        out_specs=pl.BlockSpec((400, D), lambda i: (i, 0)),
        out_shape=jax.ShapeDtypeStruct((N, D), jnp.float32),
    )(pb, pb, pa, norm, s1, wr1)


_SC_MESH = plsc.VectorSubcoreMesh(core_axis_name="c", subcore_axis_name="s")


def _make_sc_scatter(cpt):
    @functools.partial(
        pl.kernel,
        mesh=_SC_MESH,
        out_type=jax.ShapeDtypeStruct((2, NPAD, D), jnp.float32),
        scratch_types=[
            pltpu.VMEM((CH,), jnp.int32),
            pltpu.VMEM((CH,), jnp.int32),
            pltpu.VMEM((CH, D), jnp.float32),
            pltpu.VMEM_SHARED((NPAD, D), jnp.float32),
            pltpu.SemaphoreType.DMA,
        ],
    )
    def _sc_scatter(g_hbm, gidx_hbm, dst_hbm, out_hbm,
                    idx_v, dst_v, rows_v, acc_sh, sem):
        cid = lax.axis_index("c")
        sid = lax.axis_index("s")
        wid = cid * 16 + sid

        def zfill(i, carry):
            r = i // 8
            c = (i % 8) * 16
            rows_v[r, pl.ds(c, 16)] = jnp.zeros((16,), jnp.float32)
            return carry
        lax.fori_loop(0, ROW_CHUNK * 8, zfill, 0)
        rowbase = sid * ROWS_PER_TILE
        zsrc = rows_v.at[pl.ds(0, ROW_CHUNK)]

        def zcopy(k, carry):
            pltpu.sync_copy(zsrc, acc_sh.at[pl.ds(rowbase + k * ROW_CHUNK, ROW_CHUNK)])
            return carry
        lax.fori_loop(0, ROWS_PER_TILE // ROW_CHUNK, zcopy, 0)
        plsc.subcore_barrier()

        ebase = wid * cpt * CH

        def body(i, carry):
            b = ebase + i * CH
            pltpu.sync_copy(gidx_hbm.at[pl.ds(b, CH)], idx_v)
            pltpu.sync_copy(dst_hbm.at[pl.ds(b, CH)], dst_v)
            pltpu.async_copy(g_hbm.at[idx_v], rows_v, sem).wait()
            pltpu.sync_copy(rows_v, acc_sh.at[dst_v], add=True)
            return carry
        lax.fori_loop(0, cpt, body, 0)
        plsc.subcore_barrier()

        def ocopy(k, carry):
            pltpu.sync_copy(acc_sh.at[pl.ds(rowbase + k * ROW_CHUNK, ROW_CHUNK)], zsrc)
            pltpu.sync_copy(zsrc, out_hbm.at[cid, pl.ds(rowbase + k * ROW_CHUNK, ROW_CHUNK)])
            return carry
        lax.fori_loop(0, ROWS_PER_TILE // ROW_CHUNK, ocopy, 0)

    return _sc_scatter


_sc_scatter_a = _make_sc_scatter(CPTA)
_sc_scatter_b = _make_sc_scatter(CPTB)


def kernel(ent_embed, rel_embed, norm, W_rel_0, W_loop_0, W_rel_1, W_loop_1,
           edge_index, rel_id):
    src = edge_index[0]
    dst = edge_index[1]
    pad = EPH - E
    rel_spread = rel_id + R * (jnp.arange(E, dtype=jnp.int32) % REPK)
    srcg = jnp.concatenate([src, jnp.zeros((pad,), jnp.int32)])
    relg = jnp.concatenate([rel_spread + N, jnp.zeros((pad,), jnp.int32)])
    dstp = jnp.concatenate([dst, jnp.full((pad,), N, jnp.int32)])
    gidx_a = jnp.concatenate([srcg, relg])
    ddst_a = jnp.concatenate([dstp, dstp])
    gidx_b = srcg
    ddst_b = dstp
    table_a = jnp.concatenate([ent_embed, jnp.tile(rel_embed, (REPK, 1))], axis=0)

    p_a = _sc_scatter_a(table_a, gidx_a, ddst_a)
    h1, s1 = _mid_tc(p_a, norm, ent_embed, W_rel_0, W_loop_0, W_loop_1)
    p_b = _sc_scatter_b(h1, gidx_b, ddst_b)
    return _fin_tc(p_b, p_a, norm, s1, W_rel_1)


# speedup vs baseline: 1.4363x; 1.4363x over previous
import functools

import jax
import jax.numpy as jnp
from jax import lax
from jax.experimental import pallas as pl
from jax.experimental.pallas import tpu as pltpu
from jax.experimental.pallas import tpu_sc as plsc

N = 10000
D = 128
R = 200
E = 320000
SLOPE = (1.0 / 8.0 + 1.0 / 3.0) / 2.0

NPAD = 10240
ROWS_PER_TILE = NPAD // 16
ROW_CHUNK = 128
CH = 128
NW = 32
CPTA = 158
CPTB = 79
EPH = 16 * CPTA * CH
REPK = 32


def _dotT(x, w):
    return lax.dot_general(x, w, (((1,), (1,)), ((), ())),
                           preferred_element_type=jnp.float32)


def _mid_body(pa_ref, pb_ref, norm_ref, ent_ref, wr0_ref, wl0_ref, wl1_ref,
              h1_ref, s1_ref):
    agg = pa_ref[0] + pb_ref[0]
    h1 = _dotT(agg, wr0_ref[...]) * norm_ref[...] + _dotT(ent_ref[...], wl0_ref[...])
    h1 = jnp.where(h1 >= 0, h1, SLOPE * h1)
    h1_ref[...] = h1
    s1_ref[...] = _dotT(h1, wl1_ref[...])


def _mid_tc(p, norm, ent, wr0, wl0, wl1):
    return pl.pallas_call(
        _mid_body,
        grid=(25,),
        in_specs=[
            pl.BlockSpec((1, 400, D), lambda i: (0, i, 0)),
            pl.BlockSpec((1, 400, D), lambda i: (1, i, 0)),
            pl.BlockSpec((400, 1), lambda i: (i, 0)),
            pl.BlockSpec((400, D), lambda i: (i, 0)),
            pl.BlockSpec((D, D), lambda i: (0, 0)),
            pl.BlockSpec((D, D), lambda i: (0, 0)),
            pl.BlockSpec((D, D), lambda i: (0, 0)),
        ],
        out_specs=[
            pl.BlockSpec((400, D), lambda i: (i, 0)),
            pl.BlockSpec((400, D), lambda i: (i, 0)),
        ],
        out_shape=[
            jax.ShapeDtypeStruct((N, D), jnp.float32),
            jax.ShapeDtypeStruct((N, D), jnp.float32),
        ],
    )(p, p, norm, ent, wr0, wl0, wl1)


def _fin_body(pb0_ref, pb1_ref, pa1_ref, norm_ref, s1_ref, wr1_ref, o_ref):
    agg = pb0_ref[0] + pb1_ref[0] + pa1_ref[0]
    h = _dotT(agg, wr1_ref[...]) * norm_ref[...] + s1_ref[...]
    o_ref[...] = jnp.where(h >= 0, h, SLOPE * h)


def _fin_tc(pb, pa, norm, s1, wr1):
    return pl.pallas_call(
        _fin_body,
        grid=(25,),
        in_specs=[
            pl.BlockSpec((1, 400, D), lambda i: (0, i, 0)),
            pl.BlockSpec((1, 400, D), lambda i: (1, i, 0)),
            pl.BlockSpec((1, 400, D), lambda i: (1, i, 0)),
            pl.BlockSpec((400, 1), lambda i: (i, 0)),
            pl.BlockSpec((400, D), lambda i: (i, 0)),
            pl.BlockSpec((D, D), lambda i: (0, 0)),
        ],
        out_specs=pl.BlockSpec((400, D), lambda i: (i, 0)),
        out_shape=jax.ShapeDtypeStruct((N, D), jnp.float32),
    )(pb, pb, pa, norm, s1, wr1)


_SC_MESH = plsc.VectorSubcoreMesh(core_axis_name="c", subcore_axis_name="s")


def _make_sc_scatter(cpt):
    @functools.partial(
        pl.kernel,
        mesh=_SC_MESH,
        out_type=jax.ShapeDtypeStruct((2, NPAD, D), jnp.float32),
        scratch_types=[
            pltpu.VMEM((CH,), jnp.int32),
            pltpu.VMEM((CH,), jnp.int32),
            pltpu.VMEM((CH, D), jnp.float32),
            pltpu.VMEM_SHARED((NPAD, D), jnp.float32),
            pltpu.SemaphoreType.DMA,
        ],
    )
    def _sc_scatter(g_hbm, gidx_hbm, dst_hbm, out_hbm,
                    idx_v, dst_v, rows_v, acc_sh, sem):
        cid = lax.axis_index("c")
        sid = lax.axis_index("s")
        wid = cid * 16 + sid

        def zfill(i, carry):
            r = i // 8
            c = (i % 8) * 16
            rows_v[r, pl.ds(c, 16)] = jnp.zeros((16,), jnp.float32)
            return carry
        lax.fori_loop(0, ROW_CHUNK * 8, zfill, 0)
        rowbase = sid * ROWS_PER_TILE
        zsrc = rows_v.at[pl.ds(0, ROW_CHUNK)]

        def zcopy(k, carry):
            pltpu.sync_copy(zsrc, acc_sh.at[pl.ds(rowbase + k * ROW_CHUNK, ROW_CHUNK)])
            return carry
        lax.fori_loop(0, ROWS_PER_TILE // ROW_CHUNK, zcopy, 0)
        plsc.subcore_barrier()

        ebase = wid * cpt * CH

        def body(i, carry):
            b = ebase + i * CH
            pltpu.sync_copy(gidx_hbm.at[pl.ds(b, CH)], idx_v)
            pltpu.sync_copy(dst_hbm.at[pl.ds(b, CH)], dst_v)
            pltpu.async_copy(g_hbm.at[idx_v], rows_v, sem).wait()
            pltpu.sync_copy(rows_v, acc_sh.at[dst_v], add=True)
            return carry
        lax.fori_loop(0, cpt, body, 0)
        plsc.subcore_barrier()

        def ocopy(k, carry):
            pltpu.sync_copy(acc_sh.at[pl.ds(rowbase + k * ROW_CHUNK, ROW_CHUNK)], zsrc)
            pltpu.sync_copy(zsrc, out_hbm.at[cid, pl.ds(rowbase + k * ROW_CHUNK, ROW_CHUNK)])
            return carry
        lax.fori_loop(0, ROWS_PER_TILE // ROW_CHUNK, ocopy, 0)

    return _sc_scatter


_sc_scatter_a = _make_sc_scatter(CPTA)
_sc_scatter_b = _make_sc_scatter(CPTB)


def kernel(ent_embed, rel_embed, norm, W_rel_0, W_loop_0, W_rel_1, W_loop_1,
           edge_index, rel_id):
    src = edge_index[0]
    dst = edge_index[1]
    pad = EPH - E
    rel_spread = rel_id + R * (jnp.arange(E, dtype=jnp.int32) % REPK)
    pad_dst = N + jnp.arange(pad, dtype=jnp.int32) % (NPAD - N)
    srcg = jnp.concatenate([src, jnp.zeros((pad,), jnp.int32)])
    relg = jnp.concatenate([rel_spread + N, jnp.zeros((pad,), jnp.int32)])
    dstp = jnp.concatenate([dst, pad_dst])
    gidx_a = jnp.concatenate([srcg, relg])
    ddst_a = jnp.concatenate([dstp, dstp])
    gidx_b = srcg
    ddst_b = dstp
    table_a = jnp.concatenate([ent_embed, jnp.tile(rel_embed, (REPK, 1))], axis=0)

    p_a = _sc_scatter_a(table_a, gidx_a, ddst_a)
    h1, s1 = _mid_tc(p_a, norm, ent_embed, W_rel_0, W_loop_0, W_loop_1)
    p_b = _sc_scatter_b(h1, gidx_b, ddst_b)
    return _fin_tc(p_b, p_a, norm, s1, W_rel_1)


# speedup vs baseline: 2.0101x; 1.3995x over previous
import functools

import jax
import jax.numpy as jnp
from jax import lax
from jax.experimental import pallas as pl
from jax.experimental.pallas import tpu as pltpu
from jax.experimental.pallas import tpu_sc as plsc

N = 10000
D = 128
R = 200
E = 320000
SLOPE = (1.0 / 8.0 + 1.0 / 3.0) / 2.0

NPAD = 10240
ROWS_PER_TILE = NPAD // 16
ROW_CHUNK = 128
CH = 128
NW = 32
CPTA = 158
CPTB = 79
EPH = 16 * CPTA * CH
REPK = 128


def _dotT(x, w):
    return lax.dot_general(x, w, (((1,), (1,)), ((), ())),
                           preferred_element_type=jnp.float32)


def _mid_body(pa_ref, pb_ref, norm_ref, ent_ref, wr0_ref, wl0_ref, wl1_ref,
              h1_ref, s1_ref):
    agg = pa_ref[0] + pb_ref[0]
    h1 = _dotT(agg, wr0_ref[...]) * norm_ref[...] + _dotT(ent_ref[...], wl0_ref[...])
    h1 = jnp.where(h1 >= 0, h1, SLOPE * h1)
    h1_ref[...] = h1
    s1_ref[...] = _dotT(h1, wl1_ref[...])


def _mid_tc(p, norm, ent, wr0, wl0, wl1):
    return pl.pallas_call(
        _mid_body,
        grid=(25,),
        in_specs=[
            pl.BlockSpec((1, 400, D), lambda i: (0, i, 0)),
            pl.BlockSpec((1, 400, D), lambda i: (1, i, 0)),
            pl.BlockSpec((400, 1), lambda i: (i, 0)),
            pl.BlockSpec((400, D), lambda i: (i, 0)),
            pl.BlockSpec((D, D), lambda i: (0, 0)),
            pl.BlockSpec((D, D), lambda i: (0, 0)),
            pl.BlockSpec((D, D), lambda i: (0, 0)),
        ],
        out_specs=[
            pl.BlockSpec((400, D), lambda i: (i, 0)),
            pl.BlockSpec((400, D), lambda i: (i, 0)),
        ],
        out_shape=[
            jax.ShapeDtypeStruct((N, D), jnp.float32),
            jax.ShapeDtypeStruct((N, D), jnp.float32),
        ],
    )(p, p, norm, ent, wr0, wl0, wl1)


def _fin_body(pb0_ref, pb1_ref, pa1_ref, norm_ref, s1_ref, wr1_ref, o_ref):
    agg = pb0_ref[0] + pb1_ref[0] + pa1_ref[0]
    h = _dotT(agg, wr1_ref[...]) * norm_ref[...] + s1_ref[...]
    o_ref[...] = jnp.where(h >= 0, h, SLOPE * h)


def _fin_tc(pb, pa, norm, s1, wr1):
    return pl.pallas_call(
        _fin_body,
        grid=(25,),
        in_specs=[
            pl.BlockSpec((1, 400, D), lambda i: (0, i, 0)),
            pl.BlockSpec((1, 400, D), lambda i: (1, i, 0)),
            pl.BlockSpec((1, 400, D), lambda i: (1, i, 0)),
            pl.BlockSpec((400, 1), lambda i: (i, 0)),
            pl.BlockSpec((400, D), lambda i: (i, 0)),
            pl.BlockSpec((D, D), lambda i: (0, 0)),
        ],
        out_specs=pl.BlockSpec((400, D), lambda i: (i, 0)),
        out_shape=jax.ShapeDtypeStruct((N, D), jnp.float32),
    )(pb, pb, pa, norm, s1, wr1)


_SC_MESH = plsc.VectorSubcoreMesh(core_axis_name="c", subcore_axis_name="s")


def _make_sc_scatter(cpt):
    @functools.partial(
        pl.kernel,
        mesh=_SC_MESH,
        out_type=jax.ShapeDtypeStruct((2, NPAD, D), jnp.float32),
        scratch_types=[
            pltpu.VMEM((CH,), jnp.int32),
            pltpu.VMEM((CH,), jnp.int32),
            pltpu.VMEM((CH, D), jnp.float32),
            pltpu.VMEM_SHARED((NPAD, D), jnp.float32),
            pltpu.SemaphoreType.DMA,
        ],
    )
    def _sc_scatter(g_hbm, gidx_hbm, dst_hbm, out_hbm,
                    idx_v, dst_v, rows_v, acc_sh, sem):
        cid = lax.axis_index("c")
        sid = lax.axis_index("s")
        wid = cid * 16 + sid

        def zfill(i, carry):
            r = i // 8
            c = (i % 8) * 16
            rows_v[r, pl.ds(c, 16)] = jnp.zeros((16,), jnp.float32)
            return carry
        lax.fori_loop(0, ROW_CHUNK * 8, zfill, 0)
        rowbase = sid * ROWS_PER_TILE
        zsrc = rows_v.at[pl.ds(0, ROW_CHUNK)]

        def zcopy(k, carry):
            pltpu.sync_copy(zsrc, acc_sh.at[pl.ds(rowbase + k * ROW_CHUNK, ROW_CHUNK)])
            return carry
        lax.fori_loop(0, ROWS_PER_TILE // ROW_CHUNK, zcopy, 0)
        plsc.subcore_barrier()

        ebase = wid * cpt * CH

        def body(i, carry):
            b = ebase + i * CH
            pltpu.sync_copy(gidx_hbm.at[pl.ds(b, CH)], idx_v)
            pltpu.sync_copy(dst_hbm.at[pl.ds(b, CH)], dst_v)
            pltpu.async_copy(g_hbm.at[idx_v], rows_v, sem).wait()
            pltpu.sync_copy(rows_v, acc_sh.at[dst_v], add=True)
            return carry
        lax.fori_loop(0, cpt, body, 0)
        plsc.subcore_barrier()

        def ocopy(k, carry):
            pltpu.sync_copy(acc_sh.at[pl.ds(rowbase + k * ROW_CHUNK, ROW_CHUNK)], zsrc)
            pltpu.sync_copy(zsrc, out_hbm.at[cid, pl.ds(rowbase + k * ROW_CHUNK, ROW_CHUNK)])
            return carry
        lax.fori_loop(0, ROWS_PER_TILE // ROW_CHUNK, ocopy, 0)

    return _sc_scatter


_sc_scatter_a = _make_sc_scatter(CPTA)
_sc_scatter_b = _make_sc_scatter(CPTB)


def kernel(ent_embed, rel_embed, norm, W_rel_0, W_loop_0, W_rel_1, W_loop_1,
           edge_index, rel_id):
    src = edge_index[0]
    dst = edge_index[1]
    rel_spread = rel_id + N + R * (jnp.arange(E, dtype=jnp.int32) % REPK)

    def tiled(entries, ntiles, padvals):
        per = E // ntiles
        padblk = jnp.broadcast_to(padvals, (ntiles, padvals.shape[0]))
        return jnp.concatenate([entries.reshape(ntiles, per), padblk],
                               axis=1).reshape(-1)

    pad_a = 16 * CPTA * CH // 16 - E // 16
    pad_b = 32 * CPTB * CH // 32 - E // 32
    pg_a = jnp.arange(pad_a, dtype=jnp.int32) % N
    pd_a = N + jnp.arange(pad_a, dtype=jnp.int32) % (NPAD - N)
    pg_b = jnp.arange(pad_b, dtype=jnp.int32) % N
    pd_b = N + jnp.arange(pad_b, dtype=jnp.int32) % (NPAD - N)
    gidx_a = jnp.concatenate([tiled(src, 16, pg_a), tiled(rel_spread, 16, pg_a)])
    dst_a_half = tiled(dst, 16, pd_a)
    ddst_a = jnp.concatenate([dst_a_half, dst_a_half])
    gidx_b = tiled(src, 32, pg_b)
    ddst_b = tiled(dst, 32, pd_b)
    table_a = jnp.concatenate([ent_embed, jnp.tile(rel_embed, (REPK, 1))], axis=0)

    p_a = _sc_scatter_a(table_a, gidx_a, ddst_a)
    h1, s1 = _mid_tc(p_a, norm, ent_embed, W_rel_0, W_loop_0, W_loop_1)
    p_b = _sc_scatter_b(h1, gidx_b, ddst_b)
    return _fin_tc(p_b, p_a, norm, s1, W_rel_1)


# speedup vs baseline: 2.2397x; 1.1142x over previous
import functools

import jax
import jax.numpy as jnp
from jax import lax
from jax.experimental import pallas as pl
from jax.experimental.pallas import tpu as pltpu
from jax.experimental.pallas import tpu_sc as plsc

N = 10000
D = 128
R = 200
E = 320000
SLOPE = (1.0 / 8.0 + 1.0 / 3.0) / 2.0

NPAD = 10240
ROWS_PER_TILE = NPAD // 16
ROW_CHUNK = 128
CH = 128
NW = 32
CPTA = 158
CPTB = 79
EPH = 16 * CPTA * CH
REPK = 128


def _dotT(x, w):
    return lax.dot_general(x, w, (((1,), (1,)), ((), ())),
                           preferred_element_type=jnp.float32)


def _mid_body(pa_ref, pb_ref, norm_ref, ent_ref, wr0_ref, wl0_ref, wl1_ref,
              h1_ref, s1_ref):
    agg = pa_ref[0] + pb_ref[0]
    h1 = _dotT(agg, wr0_ref[...]) * norm_ref[...] + _dotT(ent_ref[...], wl0_ref[...])
    h1 = jnp.where(h1 >= 0, h1, SLOPE * h1)
    h1_ref[...] = h1
    s1_ref[...] = _dotT(h1, wl1_ref[...])


def _mid_tc(p, norm, ent, wr0, wl0, wl1):
    return pl.pallas_call(
        _mid_body,
        grid=(25,),
        in_specs=[
            pl.BlockSpec((1, 400, D), lambda i: (0, i, 0)),
            pl.BlockSpec((1, 400, D), lambda i: (1, i, 0)),
            pl.BlockSpec((400, 1), lambda i: (i, 0)),
            pl.BlockSpec((400, D), lambda i: (i, 0)),
            pl.BlockSpec((D, D), lambda i: (0, 0)),
            pl.BlockSpec((D, D), lambda i: (0, 0)),
            pl.BlockSpec((D, D), lambda i: (0, 0)),
        ],
        out_specs=[
            pl.BlockSpec((400, D), lambda i: (i, 0)),
            pl.BlockSpec((400, D), lambda i: (i, 0)),
        ],
        out_shape=[
            jax.ShapeDtypeStruct((N, D), jnp.float32),
            jax.ShapeDtypeStruct((N, D), jnp.float32),
        ],
    )(p, p, norm, ent, wr0, wl0, wl1)


def _fin_body(pb0_ref, pb1_ref, pa1_ref, norm_ref, s1_ref, wr1_ref, o_ref):
    agg = pb0_ref[0] + pb1_ref[0] + pa1_ref[0]
    h = _dotT(agg, wr1_ref[...]) * norm_ref[...] + s1_ref[...]
    o_ref[...] = jnp.where(h >= 0, h, SLOPE * h)


def _fin_tc(pb, pa, norm, s1, wr1):
    return pl.pallas_call(
        _fin_body,
        grid=(25,),
        in_specs=[
            pl.BlockSpec((1, 400, D), lambda i: (0, i, 0)),
            pl.BlockSpec((1, 400, D), lambda i: (1, i, 0)),
            pl.BlockSpec((1, 400, D), lambda i: (1, i, 0)),
            pl.BlockSpec((400, 1), lambda i: (i, 0)),
            pl.BlockSpec((400, D), lambda i: (i, 0)),
            pl.BlockSpec((D, D), lambda i: (0, 0)),
        ],
        out_specs=pl.BlockSpec((400, D), lambda i: (i, 0)),
        out_shape=jax.ShapeDtypeStruct((N, D), jnp.float32),
    )(pb, pb, pa, norm, s1, wr1)


_SC_MESH = plsc.VectorSubcoreMesh(core_axis_name="c", subcore_axis_name="s")


def _make_sc_scatter(cpt):
    @functools.partial(
        pl.kernel,
        mesh=_SC_MESH,
        out_type=jax.ShapeDtypeStruct((2, NPAD, D), jnp.float32),
        scratch_types=[
            pltpu.VMEM((2, CH), jnp.int32),
            pltpu.VMEM((CH, D), jnp.float32),
            pltpu.VMEM_SHARED((NPAD, D), jnp.float32),
            pltpu.SemaphoreType.DMA,
        ],
    )
    def _sc_scatter(g_hbm, gd_hbm, out_hbm, gd_v, rows_v, acc_sh, sem):
        cid = lax.axis_index("c")
        sid = lax.axis_index("s")
        wid = cid * 16 + sid

        def zfill(i, carry):
            r = i // 8
            c = (i % 8) * 16
            rows_v[r, pl.ds(c, 16)] = jnp.zeros((16,), jnp.float32)
            return carry
        lax.fori_loop(0, ROW_CHUNK * 8, zfill, 0)
        rowbase = sid * ROWS_PER_TILE
        zsrc = rows_v.at[pl.ds(0, ROW_CHUNK)]

        def zcopy(k, carry):
            pltpu.sync_copy(zsrc, acc_sh.at[pl.ds(rowbase + k * ROW_CHUNK, ROW_CHUNK)])
            return carry
        lax.fori_loop(0, ROWS_PER_TILE // ROW_CHUNK, zcopy, 0)
        plsc.subcore_barrier()

        cbase = wid * cpt

        def body(i, carry):
            pltpu.sync_copy(gd_hbm.at[cbase + i], gd_v)
            pltpu.async_copy(g_hbm.at[gd_v.at[0]], rows_v, sem).wait()
            pltpu.sync_copy(rows_v, acc_sh.at[gd_v.at[1]], add=True)
            return carry
        lax.fori_loop(0, cpt, body, 0)
        plsc.subcore_barrier()

        def ocopy(k, carry):
            pltpu.sync_copy(acc_sh.at[pl.ds(rowbase + k * ROW_CHUNK, ROW_CHUNK)], zsrc)
            pltpu.sync_copy(zsrc, out_hbm.at[cid, pl.ds(rowbase + k * ROW_CHUNK, ROW_CHUNK)])
            return carry
        lax.fori_loop(0, ROWS_PER_TILE // ROW_CHUNK, ocopy, 0)

    return _sc_scatter


_sc_scatter_a = _make_sc_scatter(CPTA)
_sc_scatter_b = _make_sc_scatter(CPTB)


def kernel(ent_embed, rel_embed, norm, W_rel_0, W_loop_0, W_rel_1, W_loop_1,
           edge_index, rel_id):
    src = edge_index[0]
    dst = edge_index[1]
    rel_spread = rel_id + N + R * (jnp.arange(E, dtype=jnp.int32) % REPK)

    def tiled(entries, ntiles, padvals):
        per = E // ntiles
        padblk = jnp.broadcast_to(padvals, (ntiles, padvals.shape[0]))
        return jnp.concatenate([entries.reshape(ntiles, per), padblk],
                               axis=1).reshape(-1)

    pad_a = 16 * CPTA * CH // 16 - E // 16
    pad_b = 32 * CPTB * CH // 32 - E // 32
    pg_a = jnp.arange(pad_a, dtype=jnp.int32) % N
    pd_a = N + jnp.arange(pad_a, dtype=jnp.int32) % (NPAD - N)
    pg_b = jnp.arange(pad_b, dtype=jnp.int32) % N
    pd_b = N + jnp.arange(pad_b, dtype=jnp.int32) % (NPAD - N)
    def chunked(gidx, ddst):
        return jnp.stack([gidx.reshape(-1, CH), ddst.reshape(-1, CH)], axis=1)

    dst_a_half = tiled(dst, 16, pd_a)
    gd_a = chunked(
        jnp.concatenate([tiled(src, 16, pg_a), tiled(rel_spread, 16, pg_a)]),
        jnp.concatenate([dst_a_half, dst_a_half]))
    gd_b = chunked(tiled(src, 32, pg_b), tiled(dst, 32, pd_b))
    table_a = jnp.concatenate([ent_embed, jnp.tile(rel_embed, (REPK, 1))], axis=0)

    p_a = _sc_scatter_a(table_a, gd_a)
    h1, s1 = _mid_tc(p_a, norm, ent_embed, W_rel_0, W_loop_0, W_loop_1)
    p_b = _sc_scatter_b(h1, gd_b)
    return _fin_tc(p_b, p_a, norm, s1, W_rel_1)


# speedup vs baseline: 2.4136x; 1.0777x over previous
import functools

import jax
import jax.numpy as jnp
from jax import lax
from jax.experimental import pallas as pl
from jax.experimental.pallas import tpu as pltpu
from jax.experimental.pallas import tpu_sc as plsc

N = 10000
D = 128
R = 200
E = 320000
SLOPE = (1.0 / 8.0 + 1.0 / 3.0) / 2.0

NPAD = 10240
ROWS_PER_TILE = NPAD // 16
ROW_CHUNK = 128
CH = 128
NW = 32
CPTA = 158
CPTB = 80
EPH = 16 * CPTA * CH
REPK = 128


def _dotT(x, w):
    return lax.dot_general(x, w, (((1,), (1,)), ((), ())),
                           preferred_element_type=jnp.float32)


def _mid_body(pa_ref, pb_ref, norm_ref, ent_ref, wr0_ref, wl0_ref, wl1_ref,
              h1_ref, s1_ref):
    agg = pa_ref[0] + pb_ref[0]
    h1 = _dotT(agg, wr0_ref[...]) * norm_ref[...] + _dotT(ent_ref[...], wl0_ref[...])
    h1 = jnp.where(h1 >= 0, h1, SLOPE * h1)
    h1_ref[...] = h1
    s1_ref[...] = _dotT(h1, wl1_ref[...])


def _mid_tc(p, norm, ent, wr0, wl0, wl1):
    return pl.pallas_call(
        _mid_body,
        grid=(25,),
        in_specs=[
            pl.BlockSpec((1, 400, D), lambda i: (0, i, 0)),
            pl.BlockSpec((1, 400, D), lambda i: (1, i, 0)),
            pl.BlockSpec((400, 1), lambda i: (i, 0)),
            pl.BlockSpec((400, D), lambda i: (i, 0)),
            pl.BlockSpec((D, D), lambda i: (0, 0)),
            pl.BlockSpec((D, D), lambda i: (0, 0)),
            pl.BlockSpec((D, D), lambda i: (0, 0)),
        ],
        out_specs=[
            pl.BlockSpec((400, D), lambda i: (i, 0)),
            pl.BlockSpec((400, D), lambda i: (i, 0)),
        ],
        out_shape=[
            jax.ShapeDtypeStruct((N, D), jnp.float32),
            jax.ShapeDtypeStruct((N, D), jnp.float32),
        ],
    )(p, p, norm, ent, wr0, wl0, wl1)


def _fin_body(pb0_ref, pb1_ref, pa1_ref, norm_ref, s1_ref, wr1_ref, o_ref):
    agg = pb0_ref[0] + pb1_ref[0] + pa1_ref[0]
    h = _dotT(agg, wr1_ref[...]) * norm_ref[...] + s1_ref[...]
    o_ref[...] = jnp.where(h >= 0, h, SLOPE * h)


def _fin_tc(pb, pa, norm, s1, wr1):
    return pl.pallas_call(
        _fin_body,
        grid=(25,),
        in_specs=[
            pl.BlockSpec((1, 400, D), lambda i: (0, i, 0)),
            pl.BlockSpec((1, 400, D), lambda i: (1, i, 0)),
            pl.BlockSpec((1, 400, D), lambda i: (1, i, 0)),
            pl.BlockSpec((400, 1), lambda i: (i, 0)),
            pl.BlockSpec((400, D), lambda i: (i, 0)),
            pl.BlockSpec((D, D), lambda i: (0, 0)),
        ],
        out_specs=pl.BlockSpec((400, D), lambda i: (i, 0)),
        out_shape=jax.ShapeDtypeStruct((N, D), jnp.float32),
    )(pb, pb, pa, norm, s1, wr1)


_SC_MESH = plsc.VectorSubcoreMesh(core_axis_name="c", subcore_axis_name="s")


def _make_sc_scatter(cpt):
    @functools.partial(
        pl.kernel,
        mesh=_SC_MESH,
        out_type=jax.ShapeDtypeStruct((2, NPAD, D), jnp.float32),
        scratch_types=[
            pltpu.VMEM((2, CH), jnp.int32),
            pltpu.VMEM((2, CH), jnp.int32),
            pltpu.VMEM((CH, D), jnp.float32),
            pltpu.VMEM_SHARED((NPAD, D), jnp.float32),
            pltpu.SemaphoreType.DMA,
            pltpu.SemaphoreType.DMA,
            pltpu.SemaphoreType.DMA,
        ],
    )
    def _sc_scatter(g_hbm, gd_hbm, out_hbm, gd_v0, gd_v1, rows_v, acc_sh,
                    sem, isem0, isem1):
        cid = lax.axis_index("c")
        sid = lax.axis_index("s")
        wid = cid * 16 + sid

        def zfill(i, carry):
            r = i // 8
            c = (i % 8) * 16
            rows_v[r, pl.ds(c, 16)] = jnp.zeros((16,), jnp.float32)
            return carry
        lax.fori_loop(0, ROW_CHUNK * 8, zfill, 0)
        rowbase = sid * ROWS_PER_TILE
        zsrc = rows_v.at[pl.ds(0, ROW_CHUNK)]

        def zcopy(k, carry):
            pltpu.sync_copy(zsrc, acc_sh.at[pl.ds(rowbase + k * ROW_CHUNK, ROW_CHUNK)])
            return carry
        lax.fori_loop(0, ROWS_PER_TILE // ROW_CHUNK, zcopy, 0)
        plsc.subcore_barrier()

        cbase = wid * cpt

        def body(i, carry):
            b = cbase + 2 * i
            d0 = pltpu.async_copy(gd_hbm.at[b], gd_v0, isem0)
            d1 = pltpu.async_copy(gd_hbm.at[b + 1], gd_v1, isem1)
            d0.wait()
            pltpu.async_copy(g_hbm.at[gd_v0.at[0]], rows_v, sem).wait()
            pltpu.sync_copy(rows_v, acc_sh.at[gd_v0.at[1]], add=True)
            d1.wait()
            pltpu.async_copy(g_hbm.at[gd_v1.at[0]], rows_v, sem).wait()
            pltpu.sync_copy(rows_v, acc_sh.at[gd_v1.at[1]], add=True)
            return carry
        lax.fori_loop(0, cpt // 2, body, 0)
        plsc.subcore_barrier()

        def ocopy(k, carry):
            pltpu.sync_copy(acc_sh.at[pl.ds(rowbase + k * ROW_CHUNK, ROW_CHUNK)], zsrc)
            pltpu.sync_copy(zsrc, out_hbm.at[cid, pl.ds(rowbase + k * ROW_CHUNK, ROW_CHUNK)])
            return carry
        lax.fori_loop(0, ROWS_PER_TILE // ROW_CHUNK, ocopy, 0)

    return _sc_scatter


_sc_scatter_a = _make_sc_scatter(CPTA)
_sc_scatter_b = _make_sc_scatter(CPTB)


def kernel(ent_embed, rel_embed, norm, W_rel_0, W_loop_0, W_rel_1, W_loop_1,
           edge_index, rel_id):
    src = edge_index[0]
    dst = edge_index[1]
    rel_spread = rel_id + N + R * (jnp.arange(E, dtype=jnp.int32) % REPK)

    def tiled(entries, ntiles, padvals):
        per = E // ntiles
        padblk = jnp.broadcast_to(padvals, (ntiles, padvals.shape[0]))
        return jnp.concatenate([entries.reshape(ntiles, per), padblk],
                               axis=1).reshape(-1)

    pad_a = 16 * CPTA * CH // 16 - E // 16
    pad_b = 32 * CPTB * CH // 32 - E // 32
    pg_a = jnp.arange(pad_a, dtype=jnp.int32) % N
    pd_a = N + jnp.arange(pad_a, dtype=jnp.int32) % (NPAD - N)
    pg_b = jnp.arange(pad_b, dtype=jnp.int32) % N
    pd_b = N + jnp.arange(pad_b, dtype=jnp.int32) % (NPAD - N)
    def chunked(gidx, ddst):
        return jnp.stack([gidx.reshape(-1, CH), ddst.reshape(-1, CH)], axis=1)

    dst_a_half = tiled(dst, 16, pd_a)
    gd_a = chunked(
        jnp.concatenate([tiled(src, 16, pg_a), tiled(rel_spread, 16, pg_a)]),
        jnp.concatenate([dst_a_half, dst_a_half]))
    gd_b = chunked(tiled(src, 32, pg_b), tiled(dst, 32, pd_b))
    table_a = jnp.concatenate([ent_embed, jnp.tile(rel_embed, (REPK, 1))], axis=0)

    p_a = _sc_scatter_a(table_a, gd_a)
    h1, s1 = _mid_tc(p_a, norm, ent_embed, W_rel_0, W_loop_0, W_loop_1)
    p_b = _sc_scatter_b(h1, gd_b)
    return _fin_tc(p_b, p_a, norm, s1, W_rel_1)


# speedup vs baseline: 2.4888x; 1.0312x over previous
import functools

import jax
import jax.numpy as jnp
from jax import lax
from jax.experimental import pallas as pl
from jax.experimental.pallas import tpu as pltpu
from jax.experimental.pallas import tpu_sc as plsc

N = 10000
D = 128
R = 200
E = 320000
SLOPE = (1.0 / 8.0 + 1.0 / 3.0) / 2.0

NPAD = 10240
ROWS_PER_TILE = NPAD // 16
ROW_CHUNK = 128
CH = 128
NW = 32
CPTA = 160
CPTB = 80
EPH = 16 * CPTA * CH
REPK = 128


def _dotT(x, w):
    return lax.dot_general(x, w, (((1,), (1,)), ((), ())),
                           preferred_element_type=jnp.float32)


def _mid_body(pa_ref, pb_ref, norm_ref, ent_ref, wr0_ref, wl0_ref, wl1_ref,
              h1_ref, s1_ref):
    agg = pa_ref[0] + pb_ref[0]
    h1 = _dotT(agg, wr0_ref[...]) * norm_ref[...] + _dotT(ent_ref[...], wl0_ref[...])
    h1 = jnp.where(h1 >= 0, h1, SLOPE * h1)
    h1_ref[...] = h1
    s1_ref[...] = _dotT(h1, wl1_ref[...])


def _mid_tc(p, norm, ent, wr0, wl0, wl1):
    return pl.pallas_call(
        _mid_body,
        grid=(25,),
        in_specs=[
            pl.BlockSpec((1, 400, D), lambda i: (0, i, 0)),
            pl.BlockSpec((1, 400, D), lambda i: (1, i, 0)),
            pl.BlockSpec((400, 1), lambda i: (i, 0)),
            pl.BlockSpec((400, D), lambda i: (i, 0)),
            pl.BlockSpec((D, D), lambda i: (0, 0)),
            pl.BlockSpec((D, D), lambda i: (0, 0)),
            pl.BlockSpec((D, D), lambda i: (0, 0)),
        ],
        out_specs=[
            pl.BlockSpec((400, D), lambda i: (i, 0)),
            pl.BlockSpec((400, D), lambda i: (i, 0)),
        ],
        out_shape=[
            jax.ShapeDtypeStruct((N, D), jnp.float32),
            jax.ShapeDtypeStruct((N, D), jnp.float32),
        ],
    )(p, p, norm, ent, wr0, wl0, wl1)


def _fin_body(pb0_ref, pb1_ref, pa1_ref, norm_ref, s1_ref, wr1_ref, o_ref):
    agg = pb0_ref[0] + pb1_ref[0] + pa1_ref[0]
    h = _dotT(agg, wr1_ref[...]) * norm_ref[...] + s1_ref[...]
    o_ref[...] = jnp.where(h >= 0, h, SLOPE * h)


def _fin_tc(pb, pa, norm, s1, wr1):
    return pl.pallas_call(
        _fin_body,
        grid=(25,),
        in_specs=[
            pl.BlockSpec((1, 400, D), lambda i: (0, i, 0)),
            pl.BlockSpec((1, 400, D), lambda i: (1, i, 0)),
            pl.BlockSpec((1, 400, D), lambda i: (1, i, 0)),
            pl.BlockSpec((400, 1), lambda i: (i, 0)),
            pl.BlockSpec((400, D), lambda i: (i, 0)),
            pl.BlockSpec((D, D), lambda i: (0, 0)),
        ],
        out_specs=pl.BlockSpec((400, D), lambda i: (i, 0)),
        out_shape=jax.ShapeDtypeStruct((N, D), jnp.float32),
    )(pb, pb, pa, norm, s1, wr1)


_SC_MESH = plsc.VectorSubcoreMesh(core_axis_name="c", subcore_axis_name="s")


def _make_sc_scatter(cpt):
    @functools.partial(
        pl.kernel,
        mesh=_SC_MESH,
        out_type=jax.ShapeDtypeStruct((2, NPAD, D), jnp.float32),
        scratch_types=[
            pltpu.VMEM((2, CH), jnp.int32),
            pltpu.VMEM((2, CH), jnp.int32),
            pltpu.VMEM((2, CH), jnp.int32),
            pltpu.VMEM((2, CH), jnp.int32),
            pltpu.VMEM((CH, D), jnp.float32),
            pltpu.VMEM_SHARED((NPAD, D), jnp.float32),
            pltpu.SemaphoreType.DMA,
            pltpu.SemaphoreType.DMA,
            pltpu.SemaphoreType.DMA,
            pltpu.SemaphoreType.DMA,
            pltpu.SemaphoreType.DMA,
        ],
    )
    def _sc_scatter(g_hbm, gd_hbm, out_hbm, gd_v0, gd_v1, gd_v2, gd_v3,
                    rows_v, acc_sh, sem, isem0, isem1, isem2, isem3):
        cid = lax.axis_index("c")
        sid = lax.axis_index("s")
        wid = cid * 16 + sid

        def zfill(i, carry):
            r = i // 8
            c = (i % 8) * 16
            rows_v[r, pl.ds(c, 16)] = jnp.zeros((16,), jnp.float32)
            return carry
        lax.fori_loop(0, ROW_CHUNK * 8, zfill, 0)
        rowbase = sid * ROWS_PER_TILE
        zsrc = rows_v.at[pl.ds(0, ROW_CHUNK)]

        def zcopy(k, carry):
            pltpu.sync_copy(zsrc, acc_sh.at[pl.ds(rowbase + k * ROW_CHUNK, ROW_CHUNK)])
            return carry
        lax.fori_loop(0, ROWS_PER_TILE // ROW_CHUNK, zcopy, 0)
        plsc.subcore_barrier()

        cbase = wid * cpt
        gd_bufs = (gd_v0, gd_v1, gd_v2, gd_v3)
        isems = (isem0, isem1, isem2, isem3)

        def body(i, carry):
            b = cbase + 4 * i
            ds = [pltpu.async_copy(gd_hbm.at[b + j], gd_bufs[j], isems[j])
                  for j in range(4)]
            for j in range(4):
                ds[j].wait()
                pltpu.async_copy(g_hbm.at[gd_bufs[j].at[0]], rows_v, sem).wait()
                pltpu.sync_copy(rows_v, acc_sh.at[gd_bufs[j].at[1]], add=True)
            return carry
        lax.fori_loop(0, cpt // 4, body, 0)
        plsc.subcore_barrier()

        def ocopy(k, carry):
            pltpu.sync_copy(acc_sh.at[pl.ds(rowbase + k * ROW_CHUNK, ROW_CHUNK)], zsrc)
            pltpu.sync_copy(zsrc, out_hbm.at[cid, pl.ds(rowbase + k * ROW_CHUNK, ROW_CHUNK)])
            return carry
        lax.fori_loop(0, ROWS_PER_TILE // ROW_CHUNK, ocopy, 0)

    return _sc_scatter


_sc_scatter_a = _make_sc_scatter(CPTA)
_sc_scatter_b = _make_sc_scatter(CPTB)


def kernel(ent_embed, rel_embed, norm, W_rel_0, W_loop_0, W_rel_1, W_loop_1,
           edge_index, rel_id):
    src = edge_index[0]
    dst = edge_index[1]
    rel_spread = rel_id + N + R * (jnp.arange(E, dtype=jnp.int32) % REPK)

    def tiled(entries, ntiles, padvals):
        per = E // ntiles
        padblk = jnp.broadcast_to(padvals, (ntiles, padvals.shape[0]))
        return jnp.concatenate([entries.reshape(ntiles, per), padblk],
                               axis=1).reshape(-1)

    pad_a = 16 * CPTA * CH // 16 - E // 16
    pad_b = 32 * CPTB * CH // 32 - E // 32
    pg_a = jnp.arange(pad_a, dtype=jnp.int32) % N
    pd_a = N + jnp.arange(pad_a, dtype=jnp.int32) % (NPAD - N)
    pg_b = jnp.arange(pad_b, dtype=jnp.int32) % N
    pd_b = N + jnp.arange(pad_b, dtype=jnp.int32) % (NPAD - N)
    def chunked(gidx, ddst):
        return jnp.stack([gidx.reshape(-1, CH), ddst.reshape(-1, CH)], axis=1)

    dst_a_half = tiled(dst, 16, pd_a)
    gd_a = chunked(
        jnp.concatenate([tiled(src, 16, pg_a), tiled(rel_spread, 16, pg_a)]),
        jnp.concatenate([dst_a_half, dst_a_half]))
    gd_b = chunked(tiled(src, 32, pg_b), tiled(dst, 32, pd_b))
    table_a = jnp.concatenate([ent_embed, jnp.tile(rel_embed, (REPK, 1))], axis=0)

    p_a = _sc_scatter_a(table_a, gd_a)
    h1, s1 = _mid_tc(p_a, norm, ent_embed, W_rel_0, W_loop_0, W_loop_1)
    p_b = _sc_scatter_b(h1, gd_b)
    return _fin_tc(p_b, p_a, norm, s1, W_rel_1)
